# baseline (speedup 1.0000x reference)
import jax
import jax.numpy as jnp
from jax import lax
from jax.experimental import pallas as pl
from jax.experimental.pallas import tpu as pltpu
from jax.experimental.pallas import tpu_sc as plsc

EPS = 1e-10
B = 128
V = 100000
BV = 8192
NV = (V + BV - 1) // BV

NC, NS = 2, 16
NW = NC * NS
TOTAL = V * B
PER_W = TOTAL // NW
CHUNK = 40_000
NCHUNK = PER_W // CHUNK


def _zero_body(out_ref, zbuf, sem):
    wid = lax.axis_index("s") * NC + lax.axis_index("c")

    def zinit(i, carry):
        zbuf[pl.ds(i * 16, 16)] = jnp.zeros((16,), jnp.float32)
        return carry

    lax.fori_loop(0, CHUNK // 16, zinit, 0)
    base = wid * PER_W
    copies = [
        pltpu.make_async_copy(zbuf, out_ref.at[pl.ds(base + j * CHUNK, CHUNK)], sem)
        for j in range(NCHUNK)
    ]
    for c in copies:
        c.start()
    for c in copies:
        c.wait()


def _sc_zero_fill():
    f = pl.kernel(
        _zero_body,
        out_type=jax.ShapeDtypeStruct((TOTAL,), jnp.float32),
        mesh=plsc.VectorSubcoreMesh(core_axis_name="c", subcore_axis_name="s"),
        scratch_types=[
            pltpu.VMEM((CHUNK,), jnp.float32),
            pltpu.SemaphoreType.DMA,
        ],
    )
    return f()


def _reduce_body(lt_ref, ut_ref, idx_ref, m_ref):
    i = pl.program_id(0)
    z = lt_ref[...] - jnp.log(-jnp.log(ut_ref[...] + EPS) + EPS)
    row = jax.lax.broadcasted_iota(jnp.int32, z.shape, 0) + i * BV
    z = jnp.where(row < V, z, -jnp.inf)
    bmax = jnp.max(z, axis=0, keepdims=True)
    bidx = jnp.min(jnp.where(z == bmax, row, V), axis=0, keepdims=True)

    @pl.when(i == 0)
    def _():
        m_ref[...] = bmax
        idx_ref[...] = bidx

    @pl.when(i != 0)
    def _():
        better = bmax > m_ref[...]
        m_ref[...] = jnp.where(better, bmax, m_ref[...])
        idx_ref[...] = jnp.where(better, bidx, idx_ref[...])


def _scatter_body(idx_s, idxv_ref, zin_ref, out_ref, w_ref, sem):
    del zin_ref
    idxv = idxv_ref[...]

    def build(r, carry):
        w_ref[pl.ds(r, 1), :] = (idxv == idx_s[r]).astype(jnp.float32)
        return carry

    lax.fori_loop(0, B, build, 0)

    def fire(r, carry):
        j = idx_s[r]
        pltpu.make_async_copy(
            w_ref.at[pl.ds(r, 1), :], out_ref.at[pl.ds(j, 1), :], sem
        ).start()
        return carry

    lax.fori_loop(0, B, fire, 0)
    pltpu.make_async_copy(out_ref.at[pl.ds(0, B), :], w_ref, sem).wait()


def kernel(logits, u):
    z = _sc_zero_fill().reshape(V, B)
    idx = pl.pallas_call(
        _reduce_body,
        grid=(NV,),
        in_specs=[
            pl.BlockSpec((BV, B), lambda i: (i, 0)),
            pl.BlockSpec((BV, B), lambda i: (i, 0)),
        ],
        out_specs=pl.BlockSpec((1, B), lambda i: (0, 0)),
        out_shape=jax.ShapeDtypeStruct((1, B), jnp.int32),
        scratch_shapes=[pltpu.VMEM((1, B), jnp.float32)],
    )(logits.T, u.T)

    out_t = pl.pallas_call(
        _scatter_body,
        grid_spec=pltpu.PrefetchScalarGridSpec(
            num_scalar_prefetch=1,
            grid=(1,),
            in_specs=[
                pl.BlockSpec((1, B), lambda i, s: (0, 0)),
                pl.BlockSpec(memory_space=pl.ANY),
            ],
            out_specs=pl.BlockSpec(memory_space=pl.ANY),
            scratch_shapes=[
                pltpu.VMEM((B, B), jnp.float32),
                pltpu.SemaphoreType.DMA,
            ],
        ),
        out_shape=jax.ShapeDtypeStruct((V, B), jnp.float32),
        input_output_aliases={2: 0},
    )(idx.reshape(B), idx, z)
    return out_t.T


# speedup vs baseline: 1.3112x; 1.3112x over previous
import jax
import jax.numpy as jnp
from jax.experimental import pallas as pl
from jax.experimental.pallas import tpu as pltpu

EPS = 1e-10
B = 128
V = 100000
BV = 10000
NV = (V + BV - 1) // BV


def _body(lt_ref, ut_ref, out_ref, m_ref, idx_ref):
    i = pl.program_id(0)

    @pl.when(i < NV)
    def _reduce():
        z = lt_ref[...] - jnp.log(-jnp.log(ut_ref[...] + EPS) + EPS)
        row = jax.lax.broadcasted_iota(jnp.int32, z.shape, 0) + i * BV
        z = jnp.where(row < V, z, -jnp.inf)
        bmax = jnp.max(z, axis=0, keepdims=True)
        bidx = jnp.min(jnp.where(z == bmax, row, V), axis=0, keepdims=True)

        @pl.when(i == 0)
        def _():
            m_ref[...] = bmax
            idx_ref[...] = bidx

        @pl.when(i != 0)
        def _():
            better = bmax > m_ref[...]
            m_ref[...] = jnp.where(better, bmax, m_ref[...])
            idx_ref[...] = jnp.where(better, bidx, idx_ref[...])

    @pl.when(i >= NV)
    def _write():
        row = jax.lax.broadcasted_iota(jnp.int32, (BV, B), 0) + (i - NV) * BV
        out_ref[...] = (row == idx_ref[...]).astype(jnp.float32)


def kernel(logits, u):
    out_t = pl.pallas_call(
        _body,
        grid=(2 * NV,),
        in_specs=[
            pl.BlockSpec((BV, B), lambda i: (jnp.minimum(i, NV - 1), 0)),
            pl.BlockSpec((BV, B), lambda i: (jnp.minimum(i, NV - 1), 0)),
        ],
        out_specs=pl.BlockSpec((BV, B), lambda i: (jnp.maximum(i - NV, 0), 0)),
        out_shape=jax.ShapeDtypeStruct((V, B), jnp.float32),
        scratch_shapes=[
            pltpu.VMEM((1, B), jnp.float32),
            pltpu.VMEM((1, B), jnp.int32),
        ],
    )(logits.T, u.T)
    return out_t.T
